# Initial kernel scaffold; baseline (speedup 1.0000x reference)
import jax
import jax.numpy as jnp
from jax.experimental import pallas as pl


def kernel(x, edge_index, W1_l, b1_l, W1_r, W2_l, b2_l, W2_r):
    raise NotImplementedError("write your pallas kernel here")


# speedup vs baseline: 1.3974x; 1.3974x over previous
import numpy as np

import jax
import jax.numpy as jnp
from jax import lax
from jax.experimental import pallas as pl
from jax.experimental.pallas import tpu as pltpu
from jax.experimental.pallas import tpu_sc as plsc

_N = 10000
_E = 320000
_NC = 2
_NS = 16
_NW = _NC * _NS


def _build_luts():
  lut = np.zeros(256 * 8, np.int32)
  pc = np.zeros(256, np.int32)
  for b in range(256):
    bits = [i for i in range(8) if b & (1 << i)]
    pc[b] = len(bits)
    for j, p in enumerate(bits):
      lut[b * 8 + j] = p
  return lut, pc


_LUT, _PC = _build_luts()


def _make_segmax(n, f, e, k, g):
  nown = (-(-n // _NW) + 7) // 8 * 8
  n_last = n - nown * (_NW - 1)
  nchunk = e // k
  assert nchunk % 2 == 0 and k % 16 == 0 and g % 16 == 0 and f % 16 == 0
  assert 0 < n_last <= nown and g <= 128
  pcap = k + 2 * g
  assert pcap % 16 == 0

  mesh = plsc.VectorSubcoreMesh(
      core_axis_name="c", subcore_axis_name="s",
      num_cores=_NC, num_subcores=_NS)

  def body(x_hbm, src_hbm, dst_hbm, lut_hbm, pc_hbm, out_hbm,
           src_blk0, src_blk1, dst_blk0, dst_blk1, psrc, pdst, lut, pc,
           s1, s2, sp16, t16, ibuf, rowbuf, acc, sem_e, sem_g):
    src_blks = (src_blk0, src_blk1)
    dst_blks = (dst_blk0, dst_blk1)
    cid = lax.axis_index("c")
    sid = lax.axis_index("s")
    wid = sid * _NC + cid
    base = wid * nown
    lim = jnp.minimum(base + nown, n)
    nmine = lim - base
    neg = jnp.float32(-jnp.inf)

    pltpu.sync_copy(lut_hbm, lut)
    pltpu.sync_copy(pc_hbm, pc)

    jvec = lax.iota(jnp.int32, 16)
    pw2 = jnp.left_shift(jnp.int32(1), jvec)
    jlt8 = jvec < 8
    hi8c = jnp.where(jlt8, 0, 8)
    jm8 = jvec - 8
    jp8 = jvec + 8
    zero16 = jnp.zeros((16,), jnp.int32)

    @pl.loop(0, nown + 1)
    def _init(r):
      for j in range(f // 16):
        acc[r, pl.ds(j * 16, 16)] = jnp.full((16,), neg, jnp.float32)

    @pl.loop(0, pcap // 16)
    def _initp(i):
      psrc[pl.ds(i * 16, 16)] = zero16

    def start_chunk(c, b):
      pltpu.async_copy(src_hbm.at[pl.ds(c * k, k)], src_blks[b], sem_e)
      pltpu.async_copy(dst_hbm.at[pl.ds(c * k, k)], dst_blks[b], sem_e)

    def wait_chunk(b):
      pltpu.make_async_copy(src_hbm.at[pl.ds(0, k)], src_blks[b], sem_e).wait()
      pltpu.make_async_copy(dst_hbm.at[pl.ds(0, k)], dst_blks[b], sem_e).wait()

    def tree_sum(v):
      for d in (1, 2, 4, 8):
        t16[pl.ds(0, 16)] = v
        v = v + plsc.load_gather(t16, [jnp.bitwise_xor(jvec, d)])
      return v

    def compact(b, pend):
      def grp(i, pend):
        vs = src_blks[b][pl.ds(i * 16, 16)]
        vd = dst_blks[b][pl.ds(i * 16, 16)]
        vloc = vd - base
        m = (vloc >= 0) & (vloc < nmine)
        mval = tree_sum(jnp.where(m, pw2, zero16))
        mlo8 = jnp.left_shift(jnp.bitwise_and(mval, 255), 3)
        mhi8 = jnp.left_shift(jnp.right_shift(mval, 8), 3)
        idx1 = jnp.where(jlt8, mlo8 + jvec, mhi8 + jm8)
        sp = plsc.load_gather(lut, [idx1]) + hi8c
        cl_v = plsc.load_gather(pc, [jnp.bitwise_and(mval, 255)])
        idx2 = jnp.bitwise_and(jnp.where(jvec < cl_v, jvec, jp8 - cl_v), 15)
        sp16[pl.ds(0, 16)] = sp
        fperm = plsc.load_gather(sp16, [idx2])
        s1[pl.ds(0, 16)] = vs
        s2[pl.ds(0, 16)] = jnp.where(m, vloc, nown)
        cvs = plsc.load_gather(s1, [fperm])
        cvd = plsc.load_gather(s2, [fperm])
        psrc[pl.ds(pend, 16)] = cvs
        pdst[pl.ds(pend, 16)] = cvd
        return pend + plsc.all_reduce_population_count(m)[0]
      return lax.fori_loop(0, k // 16, grp, pend)

    def gather_batch(off):
      @pl.loop(0, g // 16)
      def _cpi(i):
        ibuf[pl.ds(i * 16, 16)] = psrc[pl.ds(off + i * 16, 16)]
      pltpu.async_copy(x_hbm.at[ibuf], rowbuf, sem_g).wait()

    def accum(off, cnt16):
      @pl.loop(0, cnt16 // 16)
      def _grp(u):
        vrow = pdst[pl.ds(off + u * 16, 16)]
        for i in range(16):
          row = vrow[i]
          for j in range(f // 16):
            sl = pl.ds(j * 16, 16)
            acc[row, sl] = jnp.maximum(acc[row, sl], rowbuf[u * 16 + i, sl])

    def flush(pend):
      nb = pend // g
      def fb(i, _):
        gather_batch(i * g)
        accum(i * g, g)
        return 0
      lax.fori_loop(0, nb, fb, 0)
      rem = pend - nb * g

      @pl.when(nb > 0)
      def _mv():
        @pl.loop(0, g // 16)
        def _m(i):
          psrc[pl.ds(i * 16, 16)] = psrc[pl.ds(nb * g + i * 16, 16)]
          pdst[pl.ds(i * 16, 16)] = pdst[pl.ds(nb * g + i * 16, 16)]
      return rem

    start_chunk(0, 0)

    def pair(p, pend):
      c0 = 2 * p
      wait_chunk(0)
      start_chunk(c0 + 1, 1)
      pend = flush(compact(0, pend))
      wait_chunk(1)

      @pl.when(c0 + 2 < nchunk)
      def _pf():
        start_chunk(c0 + 2, 0)
      pend = flush(compact(1, pend))
      return pend

    pend = lax.fori_loop(0, nchunk // 2, pair, 0)

    @pl.when(pend > 0)
    def _tail():
      pdst[pl.ds(pend, 16)] = jnp.full((16,), nown, jnp.int32)
      gather_batch(0)
      accum(0, ((pend + 15) // 16) * 16)

    @pl.loop(0, nown)
    def _fix(r):
      for j in range(f // 16):
        sl = pl.ds(j * 16, 16)
        v = acc[r, sl]
        acc[r, sl] = jnp.where(v == neg, jnp.float32(0.0), v)

    @pl.when(wid < _NW - 1)
    def _w0():
      pltpu.sync_copy(acc.at[pl.ds(0, nown)], out_hbm.at[pl.ds(base, nown)])

    @pl.when(wid == _NW - 1)
    def _w1():
      pltpu.sync_copy(acc.at[pl.ds(0, n_last)], out_hbm.at[pl.ds(base, n_last)])

  return pl.kernel(
      body,
      out_type=jax.ShapeDtypeStruct((n, f), jnp.float32),
      mesh=mesh,
      scratch_types=[
          pltpu.VMEM((k,), jnp.int32),
          pltpu.VMEM((k,), jnp.int32),
          pltpu.VMEM((k,), jnp.int32),
          pltpu.VMEM((k,), jnp.int32),
          pltpu.VMEM((pcap,), jnp.int32),
          pltpu.VMEM((pcap,), jnp.int32),
          pltpu.VMEM((2048,), jnp.int32),
          pltpu.VMEM((256,), jnp.int32),
          pltpu.VMEM((16,), jnp.int32),
          pltpu.VMEM((16,), jnp.int32),
          pltpu.VMEM((16,), jnp.int32),
          pltpu.VMEM((16,), jnp.int32),
          pltpu.VMEM((g,), jnp.int32),
          pltpu.VMEM((g, f), jnp.float32),
          pltpu.VMEM((nown + 1, f), jnp.float32),
          pltpu.SemaphoreType.DMA,
          pltpu.SemaphoreType.DMA,
      ],
      compiler_params=pltpu.CompilerParams(needs_layout_passes=False),
  )


def _linear(agg, xr, w_l, b_l, w_r, relu):
  nrows, fin = agg.shape
  fout = w_l.shape[1]
  blk = 1000
  assert nrows % blk == 0

  def body(a_ref, x_ref, wl_ref, b_ref, wr_ref, o_ref):
    y = jnp.dot(a_ref[...], wl_ref[...], preferred_element_type=jnp.float32)
    y = y + jnp.dot(x_ref[...], wr_ref[...], preferred_element_type=jnp.float32)
    y = y + b_ref[...]
    if relu:
      y = jnp.maximum(y, 0.0)
    o_ref[...] = y

  return pl.pallas_call(
      body,
      grid=(nrows // blk,),
      in_specs=[
          pl.BlockSpec((blk, fin), lambda i: (i, 0)),
          pl.BlockSpec((blk, fin), lambda i: (i, 0)),
          pl.BlockSpec((fin, fout), lambda i: (0, 0)),
          pl.BlockSpec((1, fout), lambda i: (0, 0)),
          pl.BlockSpec((fin, fout), lambda i: (0, 0)),
      ],
      out_specs=pl.BlockSpec((blk, fout), lambda i: (i, 0)),
      out_shape=jax.ShapeDtypeStruct((nrows, fout), jnp.float32),
  )(agg, xr, w_l, b_l.reshape(1, fout), w_r)


_SEG128 = _make_segmax(_N, 128, _E, 1600, 128)
_SEG256 = _make_segmax(_N, 256, _E, 1600, 128)


def kernel(x, edge_index, W1_l, b1_l, W1_r, W2_l, b2_l, W2_r):
  src = edge_index[0]
  dst = edge_index[1]
  agg1 = _SEG128(x, src, dst, _LUT, _PC)
  h = _linear(agg1, x, W1_l, b1_l, W1_r, relu=True)
  agg2 = _SEG256(h, src, dst, _LUT, _PC)
  return _linear(agg2, h, W2_l, b2_l, W2_r, relu=False)


# speedup vs baseline: 1.4170x; 1.0140x over previous
import numpy as np

import jax
import jax.numpy as jnp
from jax import lax
from jax.experimental import pallas as pl
from jax.experimental.pallas import tpu as pltpu
from jax.experimental.pallas import tpu_sc as plsc

_N = 10000
_E = 320000
_NC = 2
_NS = 16
_NW = _NC * _NS


def _build_luts():
  lut = np.zeros(256 * 8, np.int32)
  pc = np.zeros(256, np.int32)
  for b in range(256):
    bits = [i for i in range(8) if b & (1 << i)]
    pc[b] = len(bits)
    for j, p in enumerate(bits):
      lut[b * 8 + j] = p
  return lut, pc


_LUT, _PC = _build_luts()


def _make_segmax(n, f, e, k, g):
  nown = (-(-n // _NW) + 7) // 8 * 8
  n_last = n - nown * (_NW - 1)
  nchunk = e // k
  assert nchunk % 2 == 0 and k % 16 == 0 and g % 16 == 0 and f % 16 == 0
  assert 0 < n_last <= nown and g <= 128
  pcap = k + 2 * g
  assert pcap % 16 == 0

  mesh = plsc.VectorSubcoreMesh(
      core_axis_name="c", subcore_axis_name="s",
      num_cores=_NC, num_subcores=_NS)

  def body(x_hbm, src_hbm, dst_hbm, lut_hbm, pc_hbm, out_hbm,
           src_blk0, src_blk1, dst_blk0, dst_blk1, psrc, pdst, lut, pc,
           s1, s2, sp16, t16, ibuf, rowbuf, acc, sem_e, sem_g):
    src_blks = (src_blk0, src_blk1)
    dst_blks = (dst_blk0, dst_blk1)
    cid = lax.axis_index("c")
    sid = lax.axis_index("s")
    wid = sid * _NC + cid
    base = wid * nown
    lim = jnp.minimum(base + nown, n)
    nmine = lim - base
    neg = jnp.float32(-jnp.inf)

    pltpu.sync_copy(lut_hbm, lut)
    pltpu.sync_copy(pc_hbm, pc)

    jvec = lax.iota(jnp.int32, 16)
    pw2 = jnp.left_shift(jnp.int32(1), jvec)
    jlt8 = jvec < 8
    hi8c = jnp.where(jlt8, 0, 8)
    jm8 = jvec - 8
    jp8 = jvec + 8
    zero16 = jnp.zeros((16,), jnp.int32)

    @pl.loop(0, nown + 1)
    def _init(r):
      for j in range(f // 16):
        acc[r, pl.ds(j * 16, 16)] = jnp.full((16,), neg, jnp.float32)

    @pl.loop(0, pcap // 16)
    def _initp(i):
      psrc[pl.ds(i * 16, 16)] = zero16

    def start_chunk(c, b):
      pltpu.async_copy(src_hbm.at[pl.ds(c * k, k)], src_blks[b], sem_e)
      pltpu.async_copy(dst_hbm.at[pl.ds(c * k, k)], dst_blks[b], sem_e)

    def wait_chunk(b):
      pltpu.make_async_copy(src_hbm.at[pl.ds(0, k)], src_blks[b], sem_e).wait()
      pltpu.make_async_copy(dst_hbm.at[pl.ds(0, k)], dst_blks[b], sem_e).wait()

    def tree_sum(v):
      for d in (1, 2, 4, 8):
        t16[pl.ds(0, 16)] = v
        v = v + plsc.load_gather(t16, [jnp.bitwise_xor(jvec, d)])
      return v

    def compact(b, pend):
      def grp(i, pend):
        vs = src_blks[b][pl.ds(i * 16, 16)]
        vd = dst_blks[b][pl.ds(i * 16, 16)]
        vloc = vd - base
        m = (vloc >= 0) & (vloc < nmine)
        mval = tree_sum(jnp.where(m, pw2, zero16))
        mlo8 = jnp.left_shift(jnp.bitwise_and(mval, 255), 3)
        mhi8 = jnp.left_shift(jnp.right_shift(mval, 8), 3)
        idx1 = jnp.where(jlt8, mlo8 + jvec, mhi8 + jm8)
        sp = plsc.load_gather(lut, [idx1]) + hi8c
        cl_v = plsc.load_gather(pc, [jnp.bitwise_and(mval, 255)])
        idx2 = jnp.bitwise_and(jnp.where(jvec < cl_v, jvec, jp8 - cl_v), 15)
        sp16[pl.ds(0, 16)] = sp
        fperm = plsc.load_gather(sp16, [idx2])
        s1[pl.ds(0, 16)] = vs
        s2[pl.ds(0, 16)] = jnp.where(m, vloc, nown)
        cvs = plsc.load_gather(s1, [fperm])
        cvd = plsc.load_gather(s2, [fperm])
        psrc[pl.ds(pend, 16)] = cvs
        pdst[pl.ds(pend, 16)] = cvd
        return pend + plsc.all_reduce_population_count(m)[0]
      return lax.fori_loop(0, k // 16, grp, pend, unroll=4)

    def gather_batch(off):
      @pl.loop(0, g // 16)
      def _cpi(i):
        ibuf[pl.ds(i * 16, 16)] = psrc[pl.ds(off + i * 16, 16)]
      pltpu.async_copy(x_hbm.at[ibuf], rowbuf, sem_g).wait()

    def accum(off, cnt16):
      @pl.loop(0, cnt16 // 16)
      def _grp(u):
        vrow = pdst[pl.ds(off + u * 16, 16)]
        for i in range(16):
          row = vrow[i]
          for j in range(f // 16):
            sl = pl.ds(j * 16, 16)
            acc[row, sl] = jnp.maximum(acc[row, sl], rowbuf[u * 16 + i, sl])

    def flush(pend):
      nb = pend // g
      def fb(i, _):
        gather_batch(i * g)
        accum(i * g, g)
        return 0
      lax.fori_loop(0, nb, fb, 0)
      rem = pend - nb * g

      @pl.when(nb > 0)
      def _mv():
        @pl.loop(0, g // 16)
        def _m(i):
          psrc[pl.ds(i * 16, 16)] = psrc[pl.ds(nb * g + i * 16, 16)]
          pdst[pl.ds(i * 16, 16)] = pdst[pl.ds(nb * g + i * 16, 16)]
      return rem

    start_chunk(0, 0)

    def pair(p, pend):
      c0 = 2 * p
      wait_chunk(0)
      start_chunk(c0 + 1, 1)
      pend = flush(compact(0, pend))
      wait_chunk(1)

      @pl.when(c0 + 2 < nchunk)
      def _pf():
        start_chunk(c0 + 2, 0)
      pend = flush(compact(1, pend))
      return pend

    pend = lax.fori_loop(0, nchunk // 2, pair, 0)

    @pl.when(pend > 0)
    def _tail():
      pdst[pl.ds(pend, 16)] = jnp.full((16,), nown, jnp.int32)
      gather_batch(0)
      accum(0, ((pend + 15) // 16) * 16)

    @pl.loop(0, nown)
    def _fix(r):
      for j in range(f // 16):
        sl = pl.ds(j * 16, 16)
        v = acc[r, sl]
        acc[r, sl] = jnp.where(v == neg, jnp.float32(0.0), v)

    @pl.when(wid < _NW - 1)
    def _w0():
      pltpu.sync_copy(acc.at[pl.ds(0, nown)], out_hbm.at[pl.ds(base, nown)])

    @pl.when(wid == _NW - 1)
    def _w1():
      pltpu.sync_copy(acc.at[pl.ds(0, n_last)], out_hbm.at[pl.ds(base, n_last)])

  return pl.kernel(
      body,
      out_type=jax.ShapeDtypeStruct((n, f), jnp.float32),
      mesh=mesh,
      scratch_types=[
          pltpu.VMEM((k,), jnp.int32),
          pltpu.VMEM((k,), jnp.int32),
          pltpu.VMEM((k,), jnp.int32),
          pltpu.VMEM((k,), jnp.int32),
          pltpu.VMEM((pcap,), jnp.int32),
          pltpu.VMEM((pcap,), jnp.int32),
          pltpu.VMEM((2048,), jnp.int32),
          pltpu.VMEM((256,), jnp.int32),
          pltpu.VMEM((16,), jnp.int32),
          pltpu.VMEM((16,), jnp.int32),
          pltpu.VMEM((16,), jnp.int32),
          pltpu.VMEM((16,), jnp.int32),
          pltpu.VMEM((g,), jnp.int32),
          pltpu.VMEM((g, f), jnp.float32),
          pltpu.VMEM((nown + 1, f), jnp.float32),
          pltpu.SemaphoreType.DMA,
          pltpu.SemaphoreType.DMA,
      ],
      compiler_params=pltpu.CompilerParams(needs_layout_passes=False),
  )


def _linear(agg, xr, w_l, b_l, w_r, relu):
  nrows, fin = agg.shape
  fout = w_l.shape[1]
  blk = 1000
  assert nrows % blk == 0

  def body(a_ref, x_ref, wl_ref, b_ref, wr_ref, o_ref):
    y = jnp.dot(a_ref[...], wl_ref[...], preferred_element_type=jnp.float32)
    y = y + jnp.dot(x_ref[...], wr_ref[...], preferred_element_type=jnp.float32)
    y = y + b_ref[...]
    if relu:
      y = jnp.maximum(y, 0.0)
    o_ref[...] = y

  return pl.pallas_call(
      body,
      grid=(nrows // blk,),
      in_specs=[
          pl.BlockSpec((blk, fin), lambda i: (i, 0)),
          pl.BlockSpec((blk, fin), lambda i: (i, 0)),
          pl.BlockSpec((fin, fout), lambda i: (0, 0)),
          pl.BlockSpec((1, fout), lambda i: (0, 0)),
          pl.BlockSpec((fin, fout), lambda i: (0, 0)),
      ],
      out_specs=pl.BlockSpec((blk, fout), lambda i: (i, 0)),
      out_shape=jax.ShapeDtypeStruct((nrows, fout), jnp.float32),
  )(agg, xr, w_l, b_l.reshape(1, fout), w_r)


_SEG128 = _make_segmax(_N, 128, _E, 1600, 128)
_SEG256 = _make_segmax(_N, 256, _E, 1600, 128)


def kernel(x, edge_index, W1_l, b1_l, W1_r, W2_l, b2_l, W2_r):
  src = edge_index[0]
  dst = edge_index[1]
  agg1 = _SEG128(x, src, dst, _LUT, _PC)
  h = _linear(agg1, x, W1_l, b1_l, W1_r, relu=True)
  agg2 = _SEG256(h, src, dst, _LUT, _PC)
  return _linear(agg2, h, W2_l, b2_l, W2_r, relu=False)


# speedup vs baseline: 2.1949x; 1.5490x over previous
import numpy as np

import jax
import jax.numpy as jnp
from jax import lax
from jax.experimental import pallas as pl
from jax.experimental.pallas import tpu as pltpu
from jax.experimental.pallas import tpu_sc as plsc

_N = 10000
_E = 320000
_NC = 2
_NS = 16
_NW = _NC * _NS
_CU = 4


def _build_luts():
  lut = np.zeros(256 * 8, np.int32)
  for b in range(256):
    bits = [i for i in range(8) if b & (1 << i)]
    for j, p in enumerate(bits):
      lut[b * 8 + j] = p
  return lut


_LUT = _build_luts()


def _partition(n):
  nown = (-(-n // _NW) + 7) // 8 * 8
  n_last = n - nown * (_NW - 1)
  assert 0 < n_last <= nown
  return nown, n_last


def _worker_id():
  return lax.axis_index("s") * _NC + lax.axis_index("c")


def _mesh():
  return plsc.VectorSubcoreMesh(
      core_axis_name="c", subcore_axis_name="s",
      num_cores=_NC, num_subcores=_NS)


def _accum_batch(acc, dref, rref, off, cnt16, f):
  @pl.loop(0, cnt16 // 16)
  def _grp(u):
    vrow = dref[pl.ds(off + u * 16, 16)]
    for i in range(16):
      row = vrow[i]
      for j in range(f // 16):
        sl = pl.ds(j * 16, 16)
        acc[row, sl] = jnp.maximum(acc[row, sl], rref[u * 16 + i, sl])


def _init_fix_write(acc, out_hbm, wid, base, nown, n_last, f):
  neg = jnp.float32(-jnp.inf)

  @pl.loop(0, nown)
  def _fix(r):
    for j in range(f // 16):
      sl = pl.ds(j * 16, 16)
      v = acc[r, sl]
      acc[r, sl] = jnp.where(v == neg, jnp.float32(0.0), v)

  @pl.when(wid < _NW - 1)
  def _w0():
    pltpu.sync_copy(acc.at[pl.ds(0, nown)], out_hbm.at[pl.ds(base, nown)])

  @pl.when(wid == _NW - 1)
  def _w1():
    pltpu.sync_copy(acc.at[pl.ds(0, n_last)], out_hbm.at[pl.ds(base, n_last)])


def _make_segmax_scan(n, f, e, k, g):
  nown, n_last = _partition(n)
  nchunk = e // k
  cap = e + g
  assert nchunk % 2 == 0 and k % (16 * _CU) == 0 and g % 16 == 0
  assert f % 16 == 0 and g <= 128 and k % 8 == 0
  pcap = k + 2 * g
  assert pcap % 16 == 0

  def body(x_hbm, src_hbm, dst_hbm, lut_hbm,
           out_hbm, csrc_hbm, cdst_hbm, counts_hbm,
           src_blk0, src_blk1, dst_blk0, dst_blk1, psrc, pdst, lut,
           *rest):
    stages = tuple(tuple(rest[4 * u:4 * u + 4]) for u in range(_CU))
    ibuf, rowbuf, acc, sem_e0, sem_e1, sem_g, sem_w = rest[4 * _CU:]
    src_blks = (src_blk0, src_blk1)
    dst_blks = (dst_blk0, dst_blk1)
    sem_es = (sem_e0, sem_e1)
    wid = _worker_id()
    base = wid * nown
    lim = jnp.minimum(base + nown, n)
    nmine = lim - base
    wbase = wid * cap
    neg = jnp.float32(-jnp.inf)

    pltpu.sync_copy(lut_hbm, lut)

    jvec = lax.iota(jnp.int32, 16)
    pw2 = jnp.left_shift(jnp.int32(1), jvec)
    jlt8 = jvec < 8
    hi8c = jnp.where(jlt8, 0, 8)
    jm8 = jvec - 8
    jp8 = jvec + 8
    zero16 = jnp.zeros((16,), jnp.int32)
    dummy16 = jnp.full((16,), nown, jnp.int32)

    @pl.loop(0, nown + 1)
    def _init(r):
      for j in range(f // 16):
        acc[r, pl.ds(j * 16, 16)] = jnp.full((16,), neg, jnp.float32)

    @pl.loop(0, pcap // 16)
    def _initp(i):
      psrc[pl.ds(i * 16, 16)] = zero16

    def start_chunk(c, b):
      pltpu.async_copy(src_hbm.at[pl.ds(c * k, k)], src_blks[b], sem_es[b])
      pltpu.async_copy(dst_hbm.at[pl.ds(c * k, k)], dst_blks[b], sem_es[b])

    def wait_chunk(b):
      pltpu.make_async_copy(src_hbm.at[pl.ds(0, k)], src_blks[b], sem_es[b]).wait()
      pltpu.make_async_copy(dst_hbm.at[pl.ds(0, k)], dst_blks[b], sem_es[b]).wait()

    def compact(b, pend):
      def blk(o, pend):
        res = []
        for u in range(_CU):
          i = o * _CU + u
          t16, sp16, s1, s2 = stages[u]
          vs = src_blks[b][pl.ds(i * 16, 16)]
          vd = dst_blks[b][pl.ds(i * 16, 16)]
          vloc = vd - base
          m = (vloc >= 0) & (vloc < nmine)
          v = jnp.where(m, pw2, zero16)
          for d in (1, 2, 4):
            t16[pl.ds(0, 16)] = v
            v = v + plsc.load_gather(t16, [jnp.bitwise_xor(jvec, d)])
          half8 = jnp.where(jlt8, v, jnp.right_shift(v, 8))
          idx1 = jnp.left_shift(half8, 3) + jnp.where(jlt8, jvec, jm8)
          sp = plsc.load_gather(lut, [idx1]) + hi8c
          cl_v = plsc.all_reduce_population_count(m & jlt8)
          idx2 = jnp.bitwise_and(jnp.where(jvec < cl_v, jvec, jp8 - cl_v), 15)
          sp16[pl.ds(0, 16)] = sp
          fperm = plsc.load_gather(sp16, [idx2])
          s1[pl.ds(0, 16)] = vs
          s2[pl.ds(0, 16)] = jnp.where(m, vloc, nown)
          cvs = plsc.load_gather(s1, [fperm])
          cvd = plsc.load_gather(s2, [fperm])
          cnt = plsc.all_reduce_population_count(m)[0]
          res.append((cvs, cvd, cnt))
        for cvs, cvd, cnt in res:
          psrc[pl.ds(pend, 16)] = cvs
          pdst[pl.ds(pend, 16)] = cvd
          pend = pend + cnt
        return pend
      return lax.fori_loop(0, k // (16 * _CU), blk, pend)

    def gather_batch(off):
      @pl.loop(0, g // 16)
      def _cpi(i):
        ibuf[pl.ds(i * 16, 16)] = psrc[pl.ds(off + i * 16, 16)]
      pltpu.async_copy(x_hbm.at[ibuf], rowbuf, sem_g).wait()

    def spill_batch(off, t):
      t8 = pl.multiple_of(wbase + t, 16)
      pltpu.async_copy(psrc.at[pl.ds(off, g)],
                       csrc_hbm.at[pl.ds(t8, g)], sem_w)
      pltpu.async_copy(pdst.at[pl.ds(off, g)],
                       cdst_hbm.at[pl.ds(t8, g)], sem_w)

    def drain_spills(cnt2):
      def dr(i, _):
        pltpu.make_async_copy(psrc.at[pl.ds(0, g)],
                              csrc_hbm.at[pl.ds(0, g)], sem_w).wait()
        return 0
      lax.fori_loop(0, cnt2, dr, 0)

    def flush(carry):
      pend, tot = carry
      nb = pend // g

      def fb(i, t):
        gather_batch(i * g)
        spill_batch(i * g, t)
        _accum_batch(acc, pdst, rowbuf, i * g, g, f)
        return t + g
      tot = lax.fori_loop(0, nb, fb, tot)
      drain_spills(2 * nb)
      rem = pend - nb * g

      @pl.when(nb > 0)
      def _mv():
        @pl.loop(0, g // 16)
        def _m(i):
          psrc[pl.ds(i * 16, 16)] = psrc[pl.ds(nb * g + i * 16, 16)]
          pdst[pl.ds(i * 16, 16)] = pdst[pl.ds(nb * g + i * 16, 16)]
      return rem, tot

    start_chunk(0, 0)

    def pair(p, carry):
      c0 = 2 * p
      pend, tot = carry
      wait_chunk(0)
      start_chunk(c0 + 1, 1)
      pend, tot = flush((compact(0, pend), tot))
      wait_chunk(1)

      @pl.when(c0 + 2 < nchunk)
      def _pf():
        start_chunk(c0 + 2, 0)
      return flush((compact(1, pend), tot))

    pend, tot = lax.fori_loop(0, nchunk // 2, pair, (0, 0))

    padded = ((pend + 15) // 16) * 16

    @pl.when(pend > 0)
    def _tail():
      pdst[pl.ds(pend, 16)] = dummy16
      gather_batch(0)
      _accum_batch(acc, pdst, rowbuf, 0, padded, f)
      spill_batch(0, tot)
      drain_spills(2)

    tot = tot + jnp.where(pend > 0, padded, 0)

    @pl.loop(0, g // 16)
    def _padp(i):
      psrc[pl.ds(i * 16, 16)] = zero16
      pdst[pl.ds(i * 16, 16)] = dummy16
    spill_batch(0, tot)
    drain_spills(2)

    cbuf = stages[0][2]
    cbuf[pl.ds(0, 16)] = jnp.full((16,), tot, jnp.int32)
    pltpu.sync_copy(cbuf, counts_hbm.at[pl.ds(wid * 16, 16)])

    _init_fix_write(acc, out_hbm, wid, base, nown, n_last, f)

  return pl.kernel(
      body,
      out_type=(
          jax.ShapeDtypeStruct((n, f), jnp.float32),
          jax.ShapeDtypeStruct((_NW * cap,), jnp.int32),
          jax.ShapeDtypeStruct((_NW * cap,), jnp.int32),
          jax.ShapeDtypeStruct((_NW * 16,), jnp.int32),
      ),
      mesh=_mesh(),
      scratch_types=[
          pltpu.VMEM((k,), jnp.int32),
          pltpu.VMEM((k,), jnp.int32),
          pltpu.VMEM((k,), jnp.int32),
          pltpu.VMEM((k,), jnp.int32),
          pltpu.VMEM((pcap,), jnp.int32),
          pltpu.VMEM((pcap,), jnp.int32),
          pltpu.VMEM((2048,), jnp.int32),
      ] + [pltpu.VMEM((16,), jnp.int32) for _ in range(4 * _CU)] + [
          pltpu.VMEM((g,), jnp.int32),
          pltpu.VMEM((g, f), jnp.float32),
          pltpu.VMEM((nown + 1, f), jnp.float32),
          pltpu.SemaphoreType.DMA,
          pltpu.SemaphoreType.DMA,
          pltpu.SemaphoreType.DMA,
          pltpu.SemaphoreType.DMA,
      ],
      compiler_params=pltpu.CompilerParams(needs_layout_passes=False),
  )


def _make_segmax_pre(n, f, cap, g):
  nown, n_last = _partition(n)
  assert f % 16 == 0 and g % 16 == 0 and g <= 128

  def body(x_hbm, csrc_hbm, cdst_hbm, counts_hbm, out_hbm,
           ib0, ib1, pd0, pd1, rb0, rb1, cbuf, acc,
           sem_e0, sem_e1, sem_g0, sem_g1):
    wid = _worker_id()
    base = wid * nown
    wbase = wid * cap
    neg = jnp.float32(-jnp.inf)
    sets = ((ib0, pd0, rb0, sem_e0, sem_g0), (ib1, pd1, rb1, sem_e1, sem_g1))

    @pl.loop(0, nown + 1)
    def _init(r):
      for j in range(f // 16):
        acc[r, pl.ds(j * 16, 16)] = jnp.full((16,), neg, jnp.float32)

    pltpu.sync_copy(counts_hbm.at[pl.ds(wid * 16, 16)], cbuf)
    tot = cbuf[pl.ds(0, 16)][0]
    nbt = (tot + g - 1) // g

    def start_edges(i, s):
      ib, pd, _, sem_e, _ = s
      o8 = pl.multiple_of(wbase + i * g, 16)
      pltpu.async_copy(csrc_hbm.at[pl.ds(o8, g)], ib, sem_e)
      pltpu.async_copy(cdst_hbm.at[pl.ds(o8, g)], pd, sem_e)

    def wait_edges(s):
      ib, pd, _, sem_e, _ = s
      pltpu.make_async_copy(csrc_hbm.at[pl.ds(0, g)], ib, sem_e).wait()
      pltpu.make_async_copy(cdst_hbm.at[pl.ds(0, g)], pd, sem_e).wait()

    def start_gather(s):
      ib, _, rb, _, sem_g = s
      pltpu.async_copy(x_hbm.at[ib], rb, sem_g)

    def wait_gather(s):
      ib, _, rb, _, sem_g = s
      pltpu.make_async_copy(x_hbm.at[ib], rb, sem_g).wait()

    @pl.when(nbt > 0)
    def _pro():
      start_edges(0, sets[0])
      wait_edges(sets[0])
      start_gather(sets[0])

    @pl.when(nbt > 1)
    def _pro2():
      start_edges(1, sets[1])

    def proc(i, cur, nxt):
      @pl.when(i + 1 < nbt)
      def _nx():
        wait_edges(nxt)
        start_gather(nxt)
      wait_gather(cur)
      _accum_batch(acc, cur[1], cur[2], 0, g, f)

      @pl.when(i + 2 < nbt)
      def _pf():
        start_edges(i + 2, cur)

    def pairb(p, _):
      i0 = 2 * p

      @pl.when(i0 < nbt)
      def _a():
        proc(i0, sets[0], sets[1])

      @pl.when(i0 + 1 < nbt)
      def _b():
        proc(i0 + 1, sets[1], sets[0])
      return 0

    lax.fori_loop(0, (nbt + 1) // 2, pairb, 0)

    _init_fix_write(acc, out_hbm, wid, base, nown, n_last, f)

  return pl.kernel(
      body,
      out_type=jax.ShapeDtypeStruct((n, f), jnp.float32),
      mesh=_mesh(),
      scratch_types=[
          pltpu.VMEM((g,), jnp.int32),
          pltpu.VMEM((g,), jnp.int32),
          pltpu.VMEM((g,), jnp.int32),
          pltpu.VMEM((g,), jnp.int32),
          pltpu.VMEM((g, f), jnp.float32),
          pltpu.VMEM((g, f), jnp.float32),
          pltpu.VMEM((16,), jnp.int32),
          pltpu.VMEM((nown + 1, f), jnp.float32),
          pltpu.SemaphoreType.DMA,
          pltpu.SemaphoreType.DMA,
          pltpu.SemaphoreType.DMA,
          pltpu.SemaphoreType.DMA,
      ],
      compiler_params=pltpu.CompilerParams(needs_layout_passes=False),
  )


def _linear(agg, xr, w_l, b_l, w_r, relu):
  nrows, fin = agg.shape
  fout = w_l.shape[1]
  blk = 1000
  assert nrows % blk == 0

  def body(a_ref, x_ref, wl_ref, b_ref, wr_ref, o_ref):
    y = jnp.dot(a_ref[...], wl_ref[...], preferred_element_type=jnp.float32)
    y = y + jnp.dot(x_ref[...], wr_ref[...], preferred_element_type=jnp.float32)
    y = y + b_ref[...]
    if relu:
      y = jnp.maximum(y, 0.0)
    o_ref[...] = y

  return pl.pallas_call(
      body,
      grid=(nrows // blk,),
      in_specs=[
          pl.BlockSpec((blk, fin), lambda i: (i, 0)),
          pl.BlockSpec((blk, fin), lambda i: (i, 0)),
          pl.BlockSpec((fin, fout), lambda i: (0, 0)),
          pl.BlockSpec((1, fout), lambda i: (0, 0)),
          pl.BlockSpec((fin, fout), lambda i: (0, 0)),
      ],
      out_specs=pl.BlockSpec((blk, fout), lambda i: (i, 0)),
      out_shape=jax.ShapeDtypeStruct((nrows, fout), jnp.float32),
  )(agg, xr, w_l, b_l.reshape(1, fout), w_r)


_GA = 128
_CAP = _E + _GA
_SEG_A = _make_segmax_scan(_N, 128, _E, 1600, _GA)
_SEG_B = _make_segmax_pre(_N, 256, _CAP, 64)


def kernel(x, edge_index, W1_l, b1_l, W1_r, W2_l, b2_l, W2_r):
  src = edge_index[0]
  dst = edge_index[1]
  agg1, csrc, cdst, counts = _SEG_A(x, src, dst, _LUT)
  h = _linear(agg1, x, W1_l, b1_l, W1_r, relu=True)
  agg2 = _SEG_B(h, csrc, cdst, counts)
  return _linear(agg2, h, W2_l, b2_l, W2_r, relu=False)


# speedup vs baseline: 2.3640x; 1.0771x over previous
import numpy as np

import jax
import jax.numpy as jnp
from jax import lax
from jax.experimental import pallas as pl
from jax.experimental.pallas import tpu as pltpu
from jax.experimental.pallas import tpu_sc as plsc

_N = 10000
_E = 320000
_NC = 2
_NS = 16
_NW = _NC * _NS
_CU = 4

_NR = 640
_NR_LAST = _N - _NR * (_NS - 1)
_NO2 = 320
_NO2_LAST = _N - _NO2 * (_NW - 1)


def _build_luts():
  lut = np.zeros(256 * 8, np.int32)
  for b in range(256):
    bits = [i for i in range(8) if b & (1 << i)]
    for j, p in enumerate(bits):
      lut[b * 8 + j] = p
  return lut


_LUT = _build_luts()


def _mesh():
  return plsc.VectorSubcoreMesh(
      core_axis_name="c", subcore_axis_name="s",
      num_cores=_NC, num_subcores=_NS)


def _accum_batch(acc, dref, rref, off, cnt16, f):
  @pl.loop(0, cnt16 // 16)
  def _grp(u):
    vrow = dref[pl.ds(off + u * 16, 16)]
    for i in range(16):
      row = vrow[i]
      for j in range(f // 16):
        sl = pl.ds(j * 16, 16)
        acc[row, sl] = jnp.maximum(acc[row, sl], rref[u * 16 + i, sl])


def _vec_consts():
  jvec = lax.iota(jnp.int32, 16)
  pw2 = jnp.left_shift(jnp.int32(1), jvec)
  jlt8 = jvec < 8
  hi8c = jnp.where(jlt8, 0, 8)
  jm8 = jvec - 8
  jp8 = jvec + 8
  zero16 = jnp.zeros((16,), jnp.int32)
  return jvec, pw2, jlt8, hi8c, jm8, jp8, zero16


def _compact_block(src_ref, dst_ref, psrc, pdst, lut, stages, consts,
                   lo, nmine, dummy, pend, o):
  jvec, pw2, jlt8, hi8c, jm8, jp8, zero16 = consts
  res = []
  for u in range(_CU):
    i = o * _CU + u
    t16, sp16, s1, s2 = stages[u]
    vs = src_ref[pl.ds(i * 16, 16)]
    vd = dst_ref[pl.ds(i * 16, 16)]
    vloc = vd - lo
    m = (vloc >= 0) & (vloc < nmine)
    v = jnp.where(m, pw2, zero16)
    for d in (1, 2, 4):
      t16[pl.ds(0, 16)] = v
      v = v + plsc.load_gather(t16, [jnp.bitwise_xor(jvec, d)])
    half8 = jnp.where(jlt8, v, jnp.right_shift(v, 8))
    idx1 = jnp.left_shift(half8, 3) + jnp.where(jlt8, jvec, jm8)
    sp = plsc.load_gather(lut, [idx1]) + hi8c
    cl_v = plsc.all_reduce_population_count(m & jlt8)
    idx2 = jnp.bitwise_and(jnp.where(jvec < cl_v, jvec, jp8 - cl_v), 15)
    sp16[pl.ds(0, 16)] = sp
    fperm = plsc.load_gather(sp16, [idx2])
    s1[pl.ds(0, 16)] = vs
    s2[pl.ds(0, 16)] = jnp.where(m, vloc, dummy)
    cvs = plsc.load_gather(s1, [fperm])
    cvd = plsc.load_gather(s2, [fperm])
    cnt = plsc.all_reduce_population_count(m)[0]
    res.append((cvs, cvd, cnt))
  for cvs, cvd, cnt in res:
    psrc[pl.ds(pend, 16)] = cvs
    pdst[pl.ds(pend, 16)] = cvd
    pend = pend + cnt
  return pend


def _make_segmax_scan(n, f, e, k, g):
  eh = e // 2
  nchunk = eh // k
  cap = eh + g
  assert nchunk % 2 == 0 and k % (16 * _CU) == 0 and g % 16 == 0
  assert f % 16 == 0 and g <= 128 and k % 8 == 0 and cap % 8 == 0
  pcap = k + 2 * g
  assert pcap % 16 == 0
  nr, nr_last = _NR, _NR_LAST

  def body(x_hbm, src_hbm, dst_hbm, lut_hbm,
           out_hbm, csrc_hbm, cdst_hbm, counts_hbm,
           src_blk0, src_blk1, dst_blk0, dst_blk1, psrc, pdst, lut,
           *rest):
    stages = tuple(tuple(rest[4 * u:4 * u + 4]) for u in range(_CU))
    ibuf, rowbuf, acc, sem_e0, sem_e1, sem_g, sem_w = rest[4 * _CU:]
    src_blks = (src_blk0, src_blk1)
    dst_blks = (dst_blk0, dst_blk1)
    sem_es = (sem_e0, sem_e1)
    r = lax.axis_index("s")
    s = lax.axis_index("c")
    wid1 = r * 2 + s
    base = r * nr
    nmine = jnp.minimum(base + nr, n) - base
    ebase = s * eh
    wbase = wid1 * cap
    neg = jnp.float32(-jnp.inf)

    pltpu.sync_copy(lut_hbm, lut)
    consts = _vec_consts()
    zero16 = consts[6]
    dummy16 = jnp.full((16,), nr, jnp.int32)

    @pl.loop(0, nr + 1)
    def _init(row):
      for j in range(f // 16):
        acc[row, pl.ds(j * 16, 16)] = jnp.full((16,), neg, jnp.float32)

    @pl.loop(0, pcap // 16)
    def _initp(i):
      psrc[pl.ds(i * 16, 16)] = zero16

    def start_chunk(c, b):
      o8 = pl.multiple_of(ebase + c * k, 8)
      pltpu.async_copy(src_hbm.at[pl.ds(o8, k)], src_blks[b], sem_es[b])
      pltpu.async_copy(dst_hbm.at[pl.ds(o8, k)], dst_blks[b], sem_es[b])

    def wait_chunk(b):
      pltpu.make_async_copy(src_hbm.at[pl.ds(0, k)], src_blks[b],
                            sem_es[b]).wait()
      pltpu.make_async_copy(dst_hbm.at[pl.ds(0, k)], dst_blks[b],
                            sem_es[b]).wait()

    def compact(b, pend):
      def blk(o, pend):
        return _compact_block(src_blks[b], dst_blks[b], psrc, pdst, lut,
                              stages, consts, base, nmine, nr, pend, o)
      return lax.fori_loop(0, k // (16 * _CU), blk, pend)

    def gather_batch(off):
      @pl.loop(0, g // 16)
      def _cpi(i):
        ibuf[pl.ds(i * 16, 16)] = psrc[pl.ds(off + i * 16, 16)]
      pltpu.async_copy(x_hbm.at[ibuf], rowbuf, sem_g).wait()

    def spill_batch(off, t):
      t8 = pl.multiple_of(wbase + t, 16)
      pltpu.async_copy(psrc.at[pl.ds(off, g)],
                       csrc_hbm.at[pl.ds(t8, g)], sem_w)
      pltpu.async_copy(pdst.at[pl.ds(off, g)],
                       cdst_hbm.at[pl.ds(t8, g)], sem_w)

    def drain_spills(cnt2):
      def dr(i, _):
        pltpu.make_async_copy(psrc.at[pl.ds(0, g)],
                              csrc_hbm.at[pl.ds(0, g)], sem_w).wait()
        return 0
      lax.fori_loop(0, cnt2, dr, 0)

    def flush(carry):
      pend, tot = carry
      nb = pend // g

      def fb(i, t):
        gather_batch(i * g)
        spill_batch(i * g, t)
        _accum_batch(acc, pdst, rowbuf, i * g, g, f)
        return t + g
      tot = lax.fori_loop(0, nb, fb, tot)
      drain_spills(2 * nb)
      rem = pend - nb * g

      @pl.when(nb > 0)
      def _mv():
        @pl.loop(0, g // 16)
        def _m(i):
          psrc[pl.ds(i * 16, 16)] = psrc[pl.ds(nb * g + i * 16, 16)]
          pdst[pl.ds(i * 16, 16)] = pdst[pl.ds(nb * g + i * 16, 16)]
      return rem, tot

    start_chunk(0, 0)

    def pair(p, carry):
      c0 = 2 * p
      pend, tot = carry
      wait_chunk(0)
      start_chunk(c0 + 1, 1)
      pend, tot = flush((compact(0, pend), tot))
      wait_chunk(1)

      @pl.when(c0 + 2 < nchunk)
      def _pf():
        start_chunk(c0 + 2, 0)
      return flush((compact(1, pend), tot))

    pend, tot = lax.fori_loop(0, nchunk // 2, pair, (0, 0))

    padded = ((pend + 15) // 16) * 16

    @pl.when(pend > 0)
    def _tail():
      pdst[pl.ds(pend, 16)] = dummy16
      gather_batch(0)
      _accum_batch(acc, pdst, rowbuf, 0, padded, f)
      spill_batch(0, tot)
      drain_spills(2)

    tot = tot + jnp.where(pend > 0, padded, 0)

    @pl.loop(0, g // 16)
    def _padp(i):
      psrc[pl.ds(i * 16, 16)] = zero16
      pdst[pl.ds(i * 16, 16)] = dummy16
    spill_batch(0, tot)
    drain_spills(2)

    cbuf = stages[0][2]
    cbuf[pl.ds(0, 16)] = jnp.full((16,), tot, jnp.int32)
    pltpu.sync_copy(cbuf, counts_hbm.at[pl.ds(wid1 * 16, 16)])

    obase = s * n + base

    @pl.when(r < _NS - 1)
    def _w0():
      pltpu.sync_copy(acc.at[pl.ds(0, nr)], out_hbm.at[pl.ds(obase, nr)])

    @pl.when(r == _NS - 1)
    def _w1():
      pltpu.sync_copy(acc.at[pl.ds(0, nr_last)],
                      out_hbm.at[pl.ds(obase, nr_last)])

  return pl.kernel(
      body,
      out_type=(
          jax.ShapeDtypeStruct((2 * n, f), jnp.float32),
          jax.ShapeDtypeStruct((_NW * cap,), jnp.int32),
          jax.ShapeDtypeStruct((_NW * cap,), jnp.int32),
          jax.ShapeDtypeStruct((_NW * 16,), jnp.int32),
      ),
      mesh=_mesh(),
      scratch_types=[
          pltpu.VMEM((k,), jnp.int32),
          pltpu.VMEM((k,), jnp.int32),
          pltpu.VMEM((k,), jnp.int32),
          pltpu.VMEM((k,), jnp.int32),
          pltpu.VMEM((pcap,), jnp.int32),
          pltpu.VMEM((pcap,), jnp.int32),
          pltpu.VMEM((2048,), jnp.int32),
      ] + [pltpu.VMEM((16,), jnp.int32) for _ in range(4 * _CU)] + [
          pltpu.VMEM((g,), jnp.int32),
          pltpu.VMEM((g, f), jnp.float32),
          pltpu.VMEM((_NR + 1, f), jnp.float32),
          pltpu.SemaphoreType.DMA,
          pltpu.SemaphoreType.DMA,
          pltpu.SemaphoreType.DMA,
          pltpu.SemaphoreType.DMA,
      ],
      compiler_params=pltpu.CompilerParams(needs_layout_passes=False),
  )


def _make_segmax_lists(n, f, cap, kc, g):
  assert f % 16 == 0 and g % 16 == 0 and g <= 128
  assert kc % (16 * _CU) == 0 and kc <= 128
  pcap = kc + 2 * g
  no2, no2_last = _NO2, _NO2_LAST

  def body(x_hbm, csrc_hbm, cdst_hbm, counts_hbm, lut_hbm, out_hbm,
           src_blk0, src_blk1, dst_blk0, dst_blk1, psrc, pdst, lut,
           *rest):
    stages = tuple(tuple(rest[4 * u:4 * u + 4]) for u in range(_CU))
    cbuf, ibuf, rowbuf, acc, sem_e0, sem_e1, sem_g = rest[4 * _CU:]
    src_blks = (src_blk0, src_blk1)
    dst_blks = (dst_blk0, dst_blk1)
    sem_es = (sem_e0, sem_e1)
    wid = lax.axis_index("s") * _NC + lax.axis_index("c")
    parent = wid // 2
    hw = wid - parent * 2
    lo = hw * no2
    base = wid * no2
    nmine = jnp.minimum(base + no2, n) - base
    neg = jnp.float32(-jnp.inf)

    pltpu.sync_copy(lut_hbm, lut)
    consts = _vec_consts()
    zero16 = consts[6]
    dummy16 = jnp.full((16,), no2, jnp.int32)

    @pl.loop(0, no2 + 1)
    def _init(row):
      for j in range(f // 16):
        acc[row, pl.ds(j * 16, 16)] = jnp.full((16,), neg, jnp.float32)

    @pl.loop(0, pcap // 16)
    def _initp(i):
      psrc[pl.ds(i * 16, 16)] = zero16

    def gather_batch(off):
      @pl.loop(0, g // 16)
      def _cpi(i):
        ibuf[pl.ds(i * 16, 16)] = psrc[pl.ds(off + i * 16, 16)]
      pltpu.async_copy(x_hbm.at[ibuf], rowbuf, sem_g).wait()

    def flush(pend):
      nb = pend // g

      def fb(i, _):
        gather_batch(i * g)
        _accum_batch(acc, pdst, rowbuf, i * g, g, f)
        return 0
      lax.fori_loop(0, nb, fb, 0)
      rem = pend - nb * g

      @pl.when(nb > 0)
      def _mv():
        @pl.loop(0, g // 16)
        def _m(i):
          psrc[pl.ds(i * 16, 16)] = psrc[pl.ds(nb * g + i * 16, 16)]
          pdst[pl.ds(i * 16, 16)] = pdst[pl.ds(nb * g + i * 16, 16)]
      return rem

    def scan_list(sidx, pend):
      lid = parent * 2 + sidx
      wb = lid * cap
      pltpu.sync_copy(counts_hbm.at[pl.ds(lid * 16, 16)], cbuf)
      tot = cbuf[pl.ds(0, 16)][0]
      nck = (tot + kc - 1) // kc

      def start_chunk(c, b):
        o8 = pl.multiple_of(wb + c * kc, 8)
        pltpu.async_copy(csrc_hbm.at[pl.ds(o8, kc)], src_blks[b], sem_es[b])
        pltpu.async_copy(cdst_hbm.at[pl.ds(o8, kc)], dst_blks[b], sem_es[b])

      def wait_chunk(b):
        pltpu.make_async_copy(csrc_hbm.at[pl.ds(0, kc)], src_blks[b],
                              sem_es[b]).wait()
        pltpu.make_async_copy(cdst_hbm.at[pl.ds(0, kc)], dst_blks[b],
                              sem_es[b]).wait()

      def compact(b, pend):
        def blk(o, pend):
          return _compact_block(src_blks[b], dst_blks[b], psrc, pdst, lut,
                                stages, consts, lo, nmine, no2, pend, o)
        return lax.fori_loop(0, kc // (16 * _CU), blk, pend)

      @pl.when(nck > 0)
      def _p0():
        start_chunk(0, 0)

      def pairb(p, pend):
        c0 = 2 * p

        def half(c, b, pend):
          wait_chunk(b)

          @pl.when(c + 1 < nck)
          def _pf():
            start_chunk(c + 1, 1 - b)
          return flush(compact(b, pend))

        pend = lax.cond(c0 < nck, lambda q: half(c0, 0, q),
                        lambda q: q, pend)
        pend = lax.cond(c0 + 1 < nck, lambda q: half(c0 + 1, 1, q),
                        lambda q: q, pend)
        return pend

      return lax.fori_loop(0, (nck + 1) // 2, pairb, pend)

    pend = scan_list(0, 0)
    pend = scan_list(1, pend)

    padded = ((pend + 15) // 16) * 16

    @pl.when(pend > 0)
    def _tail():
      pdst[pl.ds(pend, 16)] = dummy16
      gather_batch(0)
      _accum_batch(acc, pdst, rowbuf, 0, padded, f)

    @pl.loop(0, no2)
    def _fix(row):
      for j in range(f // 16):
        sl = pl.ds(j * 16, 16)
        v = acc[row, sl]
        acc[row, sl] = jnp.where(v == neg, jnp.float32(0.0), v)

    @pl.when(wid < _NW - 1)
    def _w0():
      pltpu.sync_copy(acc.at[pl.ds(0, no2)], out_hbm.at[pl.ds(base, no2)])

    @pl.when(wid == _NW - 1)
    def _w1():
      pltpu.sync_copy(acc.at[pl.ds(0, no2_last)],
                      out_hbm.at[pl.ds(base, no2_last)])

  return pl.kernel(
      body,
      out_type=jax.ShapeDtypeStruct((n, f), jnp.float32),
      mesh=_mesh(),
      scratch_types=[
          pltpu.VMEM((kc,), jnp.int32),
          pltpu.VMEM((kc,), jnp.int32),
          pltpu.VMEM((kc,), jnp.int32),
          pltpu.VMEM((kc,), jnp.int32),
          pltpu.VMEM((pcap,), jnp.int32),
          pltpu.VMEM((pcap,), jnp.int32),
          pltpu.VMEM((2048,), jnp.int32),
      ] + [pltpu.VMEM((16,), jnp.int32) for _ in range(4 * _CU)] + [
          pltpu.VMEM((16,), jnp.int32),
          pltpu.VMEM((g,), jnp.int32),
          pltpu.VMEM((g, f), jnp.float32),
          pltpu.VMEM((_NO2 + 1, f), jnp.float32),
          pltpu.SemaphoreType.DMA,
          pltpu.SemaphoreType.DMA,
          pltpu.SemaphoreType.DMA,
      ],
      compiler_params=pltpu.CompilerParams(needs_layout_passes=False),
  )


def _linear_merge(pa, pb, xr, w_l, b_l, w_r, relu):
  nrows, fin = pa.shape
  fout = w_l.shape[1]
  blk = 1000
  assert nrows % blk == 0

  def body(pa_ref, pb_ref, x_ref, wl_ref, b_ref, wr_ref, o_ref):
    agg = jnp.maximum(pa_ref[...], pb_ref[...])
    agg = jnp.where(jnp.isneginf(agg), 0.0, agg)
    y = jnp.dot(agg, wl_ref[...], preferred_element_type=jnp.float32)
    y = y + jnp.dot(x_ref[...], wr_ref[...], preferred_element_type=jnp.float32)
    y = y + b_ref[...]
    if relu:
      y = jnp.maximum(y, 0.0)
    o_ref[...] = y

  return pl.pallas_call(
      body,
      grid=(nrows // blk,),
      in_specs=[
          pl.BlockSpec((blk, fin), lambda i: (i, 0)),
          pl.BlockSpec((blk, fin), lambda i: (i, 0)),
          pl.BlockSpec((blk, fin), lambda i: (i, 0)),
          pl.BlockSpec((fin, fout), lambda i: (0, 0)),
          pl.BlockSpec((1, fout), lambda i: (0, 0)),
          pl.BlockSpec((fin, fout), lambda i: (0, 0)),
      ],
      out_specs=pl.BlockSpec((blk, fout), lambda i: (i, 0)),
      out_shape=jax.ShapeDtypeStruct((nrows, fout), jnp.float32),
  )(pa, pb, xr, w_l, b_l.reshape(1, fout), w_r)


def _linear(agg, xr, w_l, b_l, w_r, relu):
  nrows, fin = agg.shape
  fout = w_l.shape[1]
  blk = 1000
  assert nrows % blk == 0

  def body(a_ref, x_ref, wl_ref, b_ref, wr_ref, o_ref):
    y = jnp.dot(a_ref[...], wl_ref[...], preferred_element_type=jnp.float32)
    y = y + jnp.dot(x_ref[...], wr_ref[...], preferred_element_type=jnp.float32)
    y = y + b_ref[...]
    if relu:
      y = jnp.maximum(y, 0.0)
    o_ref[...] = y

  return pl.pallas_call(
      body,
      grid=(nrows // blk,),
      in_specs=[
          pl.BlockSpec((blk, fin), lambda i: (i, 0)),
          pl.BlockSpec((blk, fin), lambda i: (i, 0)),
          pl.BlockSpec((fin, fout), lambda i: (0, 0)),
          pl.BlockSpec((1, fout), lambda i: (0, 0)),
          pl.BlockSpec((fin, fout), lambda i: (0, 0)),
      ],
      out_specs=pl.BlockSpec((blk, fout), lambda i: (i, 0)),
      out_shape=jax.ShapeDtypeStruct((nrows, fout), jnp.float32),
  )(agg, xr, w_l, b_l.reshape(1, fout), w_r)


_GA = 128
_CAP = _E // 2 + _GA
_SEG_A = _make_segmax_scan(_N, 128, _E, 1600, _GA)
_SEG_B = _make_segmax_lists(_N, 256, _CAP, 128, 64)


def kernel(x, edge_index, W1_l, b1_l, W1_r, W2_l, b2_l, W2_r):
  src = edge_index[0]
  dst = edge_index[1]
  aggp, csrc, cdst, counts = _SEG_A(x, src, dst, _LUT)
  h = _linear_merge(aggp[:_N], aggp[_N:], x, W1_l, b1_l, W1_r, relu=True)
  agg2 = _SEG_B(h, csrc, cdst, counts, _LUT)
  return _linear(agg2, h, W2_l, b2_l, W2_r, relu=False)


# speedup vs baseline: 2.5049x; 1.0596x over previous
import numpy as np

import jax
import jax.numpy as jnp
from jax import lax
from jax.experimental import pallas as pl
from jax.experimental.pallas import tpu as pltpu
from jax.experimental.pallas import tpu_sc as plsc

_N = 10000
_E = 320000
_NC = 2
_NS = 16
_NW = _NC * _NS
_CU = 4

_NR = 640
_NR_LAST = _N - _NR * (_NS - 1)
_NO2 = 320
_NO2_LAST = _N - _NO2 * (_NW - 1)


def _build_luts():
  lut = np.zeros(256 * 8, np.int32)
  for b in range(256):
    bits = [i for i in range(8) if b & (1 << i)]
    for j, p in enumerate(bits):
      lut[b * 8 + j] = p
  return lut


_LUT = _build_luts()


def _mesh():
  return plsc.VectorSubcoreMesh(
      core_axis_name="c", subcore_axis_name="s",
      num_cores=_NC, num_subcores=_NS)


def _accum_batch(acc, dref, rref, off, cnt16, f):
  @pl.loop(0, cnt16 // 16)
  def _grp(u):
    vrow = dref[pl.ds(off + u * 16, 16)]
    for i in range(16):
      row = vrow[i]
      for j in range(f // 16):
        sl = pl.ds(j * 16, 16)
        acc[row, sl] = jnp.maximum(acc[row, sl], rref[u * 16 + i, sl])


def _vec_consts():
  jvec = lax.iota(jnp.int32, 16)
  pw2 = jnp.left_shift(jnp.int32(1), jvec)
  jlt8 = jvec < 8
  hi8c = jnp.where(jlt8, 0, 8)
  jm8 = jvec - 8
  jp8 = jvec + 8
  zero16 = jnp.zeros((16,), jnp.int32)
  return jvec, pw2, jlt8, hi8c, jm8, jp8, zero16


def _compact_block(src_ref, dst_ref, psrc, pdst, lut, stages, consts,
                   lo, nmine, dummy, pend, o):
  jvec, pw2, jlt8, hi8c, jm8, jp8, zero16 = consts
  res = []
  for u in range(_CU):
    i = o * _CU + u
    t16, sp16, s1, s2 = stages[u]
    vs = src_ref[pl.ds(i * 16, 16)]
    vd = dst_ref[pl.ds(i * 16, 16)]
    vloc = vd - lo
    m = (vloc >= 0) & (vloc < nmine)
    v = jnp.where(m, pw2, zero16)
    for d in (1, 2, 4):
      t16[pl.ds(0, 16)] = v
      v = v + plsc.load_gather(t16, [jnp.bitwise_xor(jvec, d)])
    half8 = jnp.where(jlt8, v, jnp.right_shift(v, 8))
    idx1 = jnp.left_shift(half8, 3) + jnp.where(jlt8, jvec, jm8)
    sp = plsc.load_gather(lut, [idx1]) + hi8c
    cl_v = plsc.all_reduce_population_count(m & jlt8)
    idx2 = jnp.bitwise_and(jnp.where(jvec < cl_v, jvec, jp8 - cl_v), 15)
    sp16[pl.ds(0, 16)] = sp
    fperm = plsc.load_gather(sp16, [idx2])
    s1[pl.ds(0, 16)] = vs
    s2[pl.ds(0, 16)] = jnp.where(m, vloc, dummy)
    cvs = plsc.load_gather(s1, [fperm])
    cvd = plsc.load_gather(s2, [fperm])
    cnt = plsc.all_reduce_population_count(m)[0]
    res.append((cvs, cvd, cnt))
  for cvs, cvd, cnt in res:
    psrc[pl.ds(pend, 16)] = cvs
    pdst[pl.ds(pend, 16)] = cvd
    pend = pend + cnt
  return pend


def _make_segmax_scan(n, f, e, k, g):
  eh = e // 2
  nchunk = eh // k
  cap = eh + g
  assert nchunk % 2 == 0 and k % (16 * _CU) == 0 and g % 16 == 0
  assert f % 16 == 0 and g <= 128 and k % 8 == 0 and cap % 8 == 0
  pcap = k + 2 * g
  assert pcap % 16 == 0
  nr, nr_last = _NR, _NR_LAST

  def body(x_hbm, src_hbm, dst_hbm, lut_hbm,
           out_hbm, csrc_hbm, cdst_hbm, counts_hbm,
           src_blk0, src_blk1, dst_blk0, dst_blk1, psrc, pdst, lut,
           *rest):
    stages = tuple(tuple(rest[4 * u:4 * u + 4]) for u in range(_CU))
    ibuf, rowbuf, acc, sem_e0, sem_e1, sem_g, sem_w = rest[4 * _CU:]
    src_blks = (src_blk0, src_blk1)
    dst_blks = (dst_blk0, dst_blk1)
    sem_es = (sem_e0, sem_e1)
    r = lax.axis_index("s")
    s = lax.axis_index("c")
    wid1 = r * 2 + s
    base = r * nr
    nmine = jnp.minimum(base + nr, n) - base
    ebase = s * eh
    wbase = wid1 * cap
    neg = jnp.float32(-jnp.inf)

    pltpu.sync_copy(lut_hbm, lut)
    consts = _vec_consts()
    zero16 = consts[6]
    dummy16 = jnp.full((16,), nr, jnp.int32)

    @pl.loop(0, nr + 1)
    def _init(row):
      for j in range(f // 16):
        acc[row, pl.ds(j * 16, 16)] = jnp.full((16,), neg, jnp.float32)

    @pl.loop(0, pcap // 16)
    def _initp(i):
      psrc[pl.ds(i * 16, 16)] = zero16

    def start_chunk(c, b):
      o8 = pl.multiple_of(ebase + c * k, 8)
      pltpu.async_copy(src_hbm.at[pl.ds(o8, k)], src_blks[b], sem_es[b])
      pltpu.async_copy(dst_hbm.at[pl.ds(o8, k)], dst_blks[b], sem_es[b])

    def wait_chunk(b):
      pltpu.make_async_copy(src_hbm.at[pl.ds(0, k)], src_blks[b],
                            sem_es[b]).wait()
      pltpu.make_async_copy(dst_hbm.at[pl.ds(0, k)], dst_blks[b],
                            sem_es[b]).wait()

    def compact(b, pend):
      def blk(o, pend):
        return _compact_block(src_blks[b], dst_blks[b], psrc, pdst, lut,
                              stages, consts, base, nmine, nr, pend, o)
      return lax.fori_loop(0, k // (16 * _CU), blk, pend)

    def gather_batch(off):
      @pl.loop(0, g // 16)
      def _cpi(i):
        ibuf[pl.ds(i * 16, 16)] = psrc[pl.ds(off + i * 16, 16)]
      pltpu.async_copy(x_hbm.at[ibuf], rowbuf, sem_g).wait()

    def spill_batch(off, t):
      t8 = pl.multiple_of(wbase + t, 16)
      pltpu.async_copy(psrc.at[pl.ds(off, g)],
                       csrc_hbm.at[pl.ds(t8, g)], sem_w)
      pltpu.async_copy(pdst.at[pl.ds(off, g)],
                       cdst_hbm.at[pl.ds(t8, g)], sem_w)

    def drain_spills(cnt2):
      def dr(i, _):
        pltpu.make_async_copy(psrc.at[pl.ds(0, g)],
                              csrc_hbm.at[pl.ds(0, g)], sem_w).wait()
        return 0
      lax.fori_loop(0, cnt2, dr, 0)

    def flush(carry):
      pend, tot = carry
      nb = pend // g

      def fb(i, t):
        gather_batch(i * g)
        spill_batch(i * g, t)
        _accum_batch(acc, pdst, rowbuf, i * g, g, f)
        return t + g
      tot = lax.fori_loop(0, nb, fb, tot)
      drain_spills(2 * nb)
      rem = pend - nb * g

      @pl.when(nb > 0)
      def _mv():
        @pl.loop(0, g // 16)
        def _m(i):
          psrc[pl.ds(i * 16, 16)] = psrc[pl.ds(nb * g + i * 16, 16)]
          pdst[pl.ds(i * 16, 16)] = pdst[pl.ds(nb * g + i * 16, 16)]
      return rem, tot

    start_chunk(0, 0)

    def pair(p, carry):
      c0 = 2 * p
      pend, tot = carry
      wait_chunk(0)
      start_chunk(c0 + 1, 1)
      pend, tot = flush((compact(0, pend), tot))
      wait_chunk(1)

      @pl.when(c0 + 2 < nchunk)
      def _pf():
        start_chunk(c0 + 2, 0)
      return flush((compact(1, pend), tot))

    pend, tot = lax.fori_loop(0, nchunk // 2, pair, (0, 0))

    padded = ((pend + 15) // 16) * 16

    @pl.when(pend > 0)
    def _tail():
      pdst[pl.ds(pend, 16)] = dummy16
      gather_batch(0)
      _accum_batch(acc, pdst, rowbuf, 0, padded, f)
      spill_batch(0, tot)
      drain_spills(2)

    tot = tot + jnp.where(pend > 0, padded, 0)

    @pl.loop(0, g // 16)
    def _padp(i):
      psrc[pl.ds(i * 16, 16)] = zero16
      pdst[pl.ds(i * 16, 16)] = dummy16
    spill_batch(0, tot)
    drain_spills(2)

    cbuf = stages[0][2]
    cbuf[pl.ds(0, 16)] = jnp.full((16,), tot, jnp.int32)
    pltpu.sync_copy(cbuf, counts_hbm.at[pl.ds(wid1 * 16, 16)])

    obase = s * n + base

    @pl.when(r < _NS - 1)
    def _w0():
      pltpu.sync_copy(acc.at[pl.ds(0, nr)], out_hbm.at[pl.ds(obase, nr)])

    @pl.when(r == _NS - 1)
    def _w1():
      pltpu.sync_copy(acc.at[pl.ds(0, nr_last)],
                      out_hbm.at[pl.ds(obase, nr_last)])

  return pl.kernel(
      body,
      out_type=(
          jax.ShapeDtypeStruct((2 * n, f), jnp.float32),
          jax.ShapeDtypeStruct((_NW * cap,), jnp.int32),
          jax.ShapeDtypeStruct((_NW * cap,), jnp.int32),
          jax.ShapeDtypeStruct((_NW * 16,), jnp.int32),
      ),
      mesh=_mesh(),
      scratch_types=[
          pltpu.VMEM((k,), jnp.int32),
          pltpu.VMEM((k,), jnp.int32),
          pltpu.VMEM((k,), jnp.int32),
          pltpu.VMEM((k,), jnp.int32),
          pltpu.VMEM((pcap,), jnp.int32),
          pltpu.VMEM((pcap,), jnp.int32),
          pltpu.VMEM((2048,), jnp.int32),
      ] + [pltpu.VMEM((16,), jnp.int32) for _ in range(4 * _CU)] + [
          pltpu.VMEM((g,), jnp.int32),
          pltpu.VMEM((g, f), jnp.float32),
          pltpu.VMEM((_NR + 1, f), jnp.float32),
          pltpu.SemaphoreType.DMA,
          pltpu.SemaphoreType.DMA,
          pltpu.SemaphoreType.DMA,
          pltpu.SemaphoreType.DMA,
      ],
      compiler_params=pltpu.CompilerParams(needs_layout_passes=False),
  )


def _make_segmax_lists(n, f, cap, g):
  assert f == 256 and g % 16 == 0 and g <= 128
  fh = f // 2
  nr, nr_last = _NR, _NR_LAST

  def body(x_hbm, csrc_hbm, cdst_hbm, counts_hbm, out_hbm,
           ib0, ib1, pd0, pd1, rb0, rb1, cbuf, acc,
           sem_e0, sem_e1, sem_g0, sem_g1):
    wid = lax.axis_index("s") * _NC + lax.axis_index("c")
    parent = wid // 2
    ch = wid - parent * 2
    cbase = ch * fh
    base = parent * nr
    neg = jnp.float32(-jnp.inf)
    sets = ((ib0, pd0, rb0, sem_e0, sem_g0), (ib1, pd1, rb1, sem_e1, sem_g1))

    @pl.loop(0, nr + 1)
    def _init(row):
      for j in range(fh // 16):
        acc[row, pl.ds(j * 16, 16)] = jnp.full((16,), neg, jnp.float32)

    def accum_half(dref, rref, cnt16):
      @pl.loop(0, cnt16 // 16)
      def _grp(u):
        vrow = dref[pl.ds(u * 16, 16)]
        for i in range(16):
          row = vrow[i]
          for j in range(fh // 16):
            acc[row, pl.ds(j * 16, 16)] = jnp.maximum(
                acc[row, pl.ds(j * 16, 16)],
                rref[u * 16 + i, pl.ds(cbase + j * 16, 16)])

    def run_list(lid):
      wb = lid * cap
      pltpu.sync_copy(counts_hbm.at[pl.ds(lid * 16, 16)], cbuf)
      tot = cbuf[pl.ds(0, 16)][0]
      nbt = (tot + g - 1) // g

      def start_edges(i, s):
        ib, pd, _, sem_e, _ = s
        o8 = pl.multiple_of(wb + i * g, 16)
        pltpu.async_copy(csrc_hbm.at[pl.ds(o8, g)], ib, sem_e)
        pltpu.async_copy(cdst_hbm.at[pl.ds(o8, g)], pd, sem_e)

      def wait_edges(s):
        ib, pd, _, sem_e, _ = s
        pltpu.make_async_copy(csrc_hbm.at[pl.ds(0, g)], ib, sem_e).wait()
        pltpu.make_async_copy(cdst_hbm.at[pl.ds(0, g)], pd, sem_e).wait()

      def start_gather(s):
        ib, _, rb, _, sem_g = s
        pltpu.async_copy(x_hbm.at[ib], rb, sem_g)

      def wait_gather(s):
        ib, _, rb, _, sem_g = s
        pltpu.make_async_copy(x_hbm.at[ib], rb, sem_g).wait()

      @pl.when(nbt > 0)
      def _pro():
        start_edges(0, sets[0])
        wait_edges(sets[0])
        start_gather(sets[0])

      @pl.when(nbt > 1)
      def _pro2():
        start_edges(1, sets[1])

      def proc(i, cur, nxt):
        @pl.when(i + 1 < nbt)
        def _nx():
          wait_edges(nxt)
          start_gather(nxt)
        wait_gather(cur)
        accum_half(cur[1], cur[2], g)

        @pl.when(i + 2 < nbt)
        def _pf():
          start_edges(i + 2, cur)

      def pairb(p, _):
        i0 = 2 * p

        @pl.when(i0 < nbt)
        def _a():
          proc(i0, sets[0], sets[1])

        @pl.when(i0 + 1 < nbt)
        def _b():
          proc(i0 + 1, sets[1], sets[0])
        return 0

      lax.fori_loop(0, (nbt + 1) // 2, pairb, 0)

    run_list(parent * 2)
    run_list(parent * 2 + 1)

    @pl.loop(0, nr)
    def _fix(row):
      for j in range(fh // 16):
        sl = pl.ds(j * 16, 16)
        v = acc[row, sl]
        acc[row, sl] = jnp.where(v == neg, jnp.float32(0.0), v)

    @pl.when(parent < _NS - 1)
    def _w0():
      pltpu.sync_copy(acc.at[pl.ds(0, nr), :],
                      out_hbm.at[pl.ds(base, nr), pl.ds(cbase, fh)])

    @pl.when(parent == _NS - 1)
    def _w1():
      pltpu.sync_copy(acc.at[pl.ds(0, nr_last), :],
                      out_hbm.at[pl.ds(base, nr_last), pl.ds(cbase, fh)])

  return pl.kernel(
      body,
      out_type=jax.ShapeDtypeStruct((n, f), jnp.float32),
      mesh=_mesh(),
      scratch_types=[
          pltpu.VMEM((g,), jnp.int32),
          pltpu.VMEM((g,), jnp.int32),
          pltpu.VMEM((g,), jnp.int32),
          pltpu.VMEM((g,), jnp.int32),
          pltpu.VMEM((g, f), jnp.float32),
          pltpu.VMEM((g, f), jnp.float32),
          pltpu.VMEM((16,), jnp.int32),
          pltpu.VMEM((_NR + 1, f // 2), jnp.float32),
          pltpu.SemaphoreType.DMA,
          pltpu.SemaphoreType.DMA,
          pltpu.SemaphoreType.DMA,
          pltpu.SemaphoreType.DMA,
      ],
      compiler_params=pltpu.CompilerParams(needs_layout_passes=False),
  )


def _linear_merge(pa, pb, xr, w_l, b_l, w_r, relu):
  nrows, fin = pa.shape
  fout = w_l.shape[1]
  blk = 1000
  assert nrows % blk == 0

  def body(pa_ref, pb_ref, x_ref, wl_ref, b_ref, wr_ref, o_ref):
    agg = jnp.maximum(pa_ref[...], pb_ref[...])
    agg = jnp.where(jnp.isneginf(agg), 0.0, agg)
    y = jnp.dot(agg, wl_ref[...], preferred_element_type=jnp.float32)
    y = y + jnp.dot(x_ref[...], wr_ref[...], preferred_element_type=jnp.float32)
    y = y + b_ref[...]
    if relu:
      y = jnp.maximum(y, 0.0)
    o_ref[...] = y

  return pl.pallas_call(
      body,
      grid=(nrows // blk,),
      in_specs=[
          pl.BlockSpec((blk, fin), lambda i: (i, 0)),
          pl.BlockSpec((blk, fin), lambda i: (i, 0)),
          pl.BlockSpec((blk, fin), lambda i: (i, 0)),
          pl.BlockSpec((fin, fout), lambda i: (0, 0)),
          pl.BlockSpec((1, fout), lambda i: (0, 0)),
          pl.BlockSpec((fin, fout), lambda i: (0, 0)),
      ],
      out_specs=pl.BlockSpec((blk, fout), lambda i: (i, 0)),
      out_shape=jax.ShapeDtypeStruct((nrows, fout), jnp.float32),
  )(pa, pb, xr, w_l, b_l.reshape(1, fout), w_r)


def _linear(agg, xr, w_l, b_l, w_r, relu):
  nrows, fin = agg.shape
  fout = w_l.shape[1]
  blk = 1000
  assert nrows % blk == 0

  def body(a_ref, x_ref, wl_ref, b_ref, wr_ref, o_ref):
    y = jnp.dot(a_ref[...], wl_ref[...], preferred_element_type=jnp.float32)
    y = y + jnp.dot(x_ref[...], wr_ref[...], preferred_element_type=jnp.float32)
    y = y + b_ref[...]
    if relu:
      y = jnp.maximum(y, 0.0)
    o_ref[...] = y

  return pl.pallas_call(
      body,
      grid=(nrows // blk,),
      in_specs=[
          pl.BlockSpec((blk, fin), lambda i: (i, 0)),
          pl.BlockSpec((blk, fin), lambda i: (i, 0)),
          pl.BlockSpec((fin, fout), lambda i: (0, 0)),
          pl.BlockSpec((1, fout), lambda i: (0, 0)),
          pl.BlockSpec((fin, fout), lambda i: (0, 0)),
      ],
      out_specs=pl.BlockSpec((blk, fout), lambda i: (i, 0)),
      out_shape=jax.ShapeDtypeStruct((nrows, fout), jnp.float32),
  )(agg, xr, w_l, b_l.reshape(1, fout), w_r)


_GA = 128
_CAP = _E // 2 + _GA
_SEG_A = _make_segmax_scan(_N, 128, _E, 1600, _GA)
_SEG_B = _make_segmax_lists(_N, 256, _CAP, 64)


def kernel(x, edge_index, W1_l, b1_l, W1_r, W2_l, b2_l, W2_r):
  src = edge_index[0]
  dst = edge_index[1]
  aggp, csrc, cdst, counts = _SEG_A(x, src, dst, _LUT)
  h = _linear_merge(aggp[:_N], aggp[_N:], x, W1_l, b1_l, W1_r, relu=True)
  agg2 = _SEG_B(h, csrc, cdst, counts)
  return _linear(agg2, h, W2_l, b2_l, W2_r, relu=False)


# speedup vs baseline: 2.5086x; 1.0015x over previous
import numpy as np

import jax
import jax.numpy as jnp
from jax import lax
from jax.experimental import pallas as pl
from jax.experimental.pallas import tpu as pltpu
from jax.experimental.pallas import tpu_sc as plsc

_N = 10000
_E = 320000
_NC = 2
_NS = 16
_NW = _NC * _NS
_CU = 5

_NR = 640
_NR_LAST = _N - _NR * (_NS - 1)
_NO2 = 320
_NO2_LAST = _N - _NO2 * (_NW - 1)


def _build_luts():
  lut = np.zeros(256 * 8, np.int32)
  for b in range(256):
    bits = [i for i in range(8) if b & (1 << i)]
    for j, p in enumerate(bits):
      lut[b * 8 + j] = p
  return lut


_LUT = _build_luts()


def _mesh():
  return plsc.VectorSubcoreMesh(
      core_axis_name="c", subcore_axis_name="s",
      num_cores=_NC, num_subcores=_NS)


def _accum_batch(acc, dref, rref, off, cnt16, f):
  @pl.loop(0, cnt16 // 16)
  def _grp(u):
    vrow = dref[pl.ds(off + u * 16, 16)]
    for i in range(16):
      row = vrow[i]
      for j in range(f // 16):
        sl = pl.ds(j * 16, 16)
        acc[row, sl] = jnp.maximum(acc[row, sl], rref[u * 16 + i, sl])


def _vec_consts():
  jvec = lax.iota(jnp.int32, 16)
  pw2 = jnp.left_shift(jnp.int32(1), jvec)
  jlt8 = jvec < 8
  hi8c = jnp.where(jlt8, 0, 8)
  jm8 = jvec - 8
  jp8 = jvec + 8
  zero16 = jnp.zeros((16,), jnp.int32)
  return jvec, pw2, jlt8, hi8c, jm8, jp8, zero16


def _compact_block(src_ref, dst_ref, psrc, pdst, lut, stages, consts,
                   lo, nmine, dummy, pend, o):
  jvec, pw2, jlt8, hi8c, jm8, jp8, zero16 = consts
  res = []
  for u in range(_CU):
    i = o * _CU + u
    t16, sp16, s1, s2 = stages[u]
    vs = src_ref[pl.ds(i * 16, 16)]
    vd = dst_ref[pl.ds(i * 16, 16)]
    vloc = vd - lo
    m = (vloc >= 0) & (vloc < nmine)
    v = jnp.where(m, pw2, zero16)
    for d in (1, 2, 4):
      t16[pl.ds(0, 16)] = v
      v = v + plsc.load_gather(t16, [jnp.bitwise_xor(jvec, d)])
    half8 = jnp.where(jlt8, v, jnp.right_shift(v, 8))
    idx1 = jnp.left_shift(half8, 3) + jnp.where(jlt8, jvec, jm8)
    sp = plsc.load_gather(lut, [idx1]) + hi8c
    cl_v = plsc.all_reduce_population_count(m & jlt8)
    idx2 = jnp.bitwise_and(jnp.where(jvec < cl_v, jvec, jp8 - cl_v), 15)
    sp16[pl.ds(0, 16)] = sp
    fperm = plsc.load_gather(sp16, [idx2])
    s1[pl.ds(0, 16)] = vs
    s2[pl.ds(0, 16)] = jnp.where(m, vloc, dummy)
    cvs = plsc.load_gather(s1, [fperm])
    cvd = plsc.load_gather(s2, [fperm])
    cnt = plsc.all_reduce_population_count(m)[0]
    res.append((cvs, cvd, cnt))
  for cvs, cvd, cnt in res:
    psrc[pl.ds(pend, 16)] = cvs
    pdst[pl.ds(pend, 16)] = cvd
    pend = pend + cnt
  return pend


def _make_segmax_scan(n, f, e, k, g):
  eh = e // 2
  nchunk = eh // k
  cap = eh + g
  assert nchunk % 2 == 0 and k % (16 * _CU) == 0 and g % 16 == 0
  assert f % 16 == 0 and g <= 128 and k % 8 == 0 and cap % 8 == 0
  pcap = k + 2 * g
  assert pcap % 16 == 0
  nr, nr_last = _NR, _NR_LAST

  def body(x_hbm, src_hbm, dst_hbm, lut_hbm,
           out_hbm, csrc_hbm, cdst_hbm, counts_hbm,
           src_blk0, src_blk1, dst_blk0, dst_blk1, psrc, pdst, lut,
           *rest):
    stages = tuple(tuple(rest[4 * u:4 * u + 4]) for u in range(_CU))
    ibuf, rowbuf, acc, sem_e0, sem_e1, sem_g, sem_w = rest[4 * _CU:]
    src_blks = (src_blk0, src_blk1)
    dst_blks = (dst_blk0, dst_blk1)
    sem_es = (sem_e0, sem_e1)
    r = lax.axis_index("s")
    s = lax.axis_index("c")
    wid1 = r * 2 + s
    base = r * nr
    nmine = jnp.minimum(base + nr, n) - base
    ebase = s * eh
    wbase = wid1 * cap
    neg = jnp.float32(-jnp.inf)

    pltpu.sync_copy(lut_hbm, lut)
    consts = _vec_consts()
    zero16 = consts[6]
    dummy16 = jnp.full((16,), nr, jnp.int32)

    @pl.loop(0, nr + 1)
    def _init(row):
      for j in range(f // 16):
        acc[row, pl.ds(j * 16, 16)] = jnp.full((16,), neg, jnp.float32)

    @pl.loop(0, pcap // 16)
    def _initp(i):
      psrc[pl.ds(i * 16, 16)] = zero16

    def start_chunk(c, b):
      o8 = pl.multiple_of(ebase + c * k, 8)
      pltpu.async_copy(src_hbm.at[pl.ds(o8, k)], src_blks[b], sem_es[b])
      pltpu.async_copy(dst_hbm.at[pl.ds(o8, k)], dst_blks[b], sem_es[b])

    def wait_chunk(b):
      pltpu.make_async_copy(src_hbm.at[pl.ds(0, k)], src_blks[b],
                            sem_es[b]).wait()
      pltpu.make_async_copy(dst_hbm.at[pl.ds(0, k)], dst_blks[b],
                            sem_es[b]).wait()

    def compact(b, pend):
      def blk(o, pend):
        return _compact_block(src_blks[b], dst_blks[b], psrc, pdst, lut,
                              stages, consts, base, nmine, nr, pend, o)
      return lax.fori_loop(0, k // (16 * _CU), blk, pend)

    def gather_batch(off):
      @pl.loop(0, g // 16)
      def _cpi(i):
        ibuf[pl.ds(i * 16, 16)] = psrc[pl.ds(off + i * 16, 16)]
      pltpu.async_copy(x_hbm.at[ibuf], rowbuf, sem_g).wait()

    def spill_batch(off, t):
      t8 = pl.multiple_of(wbase + t, 16)
      pltpu.async_copy(psrc.at[pl.ds(off, g)],
                       csrc_hbm.at[pl.ds(t8, g)], sem_w)
      pltpu.async_copy(pdst.at[pl.ds(off, g)],
                       cdst_hbm.at[pl.ds(t8, g)], sem_w)

    def drain_spills(cnt2):
      def dr(i, _):
        pltpu.make_async_copy(psrc.at[pl.ds(0, g)],
                              csrc_hbm.at[pl.ds(0, g)], sem_w).wait()
        return 0
      lax.fori_loop(0, cnt2, dr, 0)

    def flush(carry):
      pend, tot = carry
      nb = pend // g

      def fb(i, t):
        gather_batch(i * g)
        spill_batch(i * g, t)
        _accum_batch(acc, pdst, rowbuf, i * g, g, f)
        return t + g
      tot = lax.fori_loop(0, nb, fb, tot)
      drain_spills(2 * nb)
      rem = pend - nb * g

      @pl.when(nb > 0)
      def _mv():
        @pl.loop(0, g // 16)
        def _m(i):
          psrc[pl.ds(i * 16, 16)] = psrc[pl.ds(nb * g + i * 16, 16)]
          pdst[pl.ds(i * 16, 16)] = pdst[pl.ds(nb * g + i * 16, 16)]
      return rem, tot

    start_chunk(0, 0)

    def pair(p, carry):
      c0 = 2 * p
      pend, tot = carry
      wait_chunk(0)
      start_chunk(c0 + 1, 1)
      pend, tot = flush((compact(0, pend), tot))
      wait_chunk(1)

      @pl.when(c0 + 2 < nchunk)
      def _pf():
        start_chunk(c0 + 2, 0)
      return flush((compact(1, pend), tot))

    pend, tot = lax.fori_loop(0, nchunk // 2, pair, (0, 0))

    padded = ((pend + 15) // 16) * 16

    @pl.when(pend > 0)
    def _tail():
      pdst[pl.ds(pend, 16)] = dummy16
      gather_batch(0)
      _accum_batch(acc, pdst, rowbuf, 0, padded, f)
      spill_batch(0, tot)
      drain_spills(2)

    tot = tot + jnp.where(pend > 0, padded, 0)

    @pl.loop(0, g // 16)
    def _padp(i):
      psrc[pl.ds(i * 16, 16)] = zero16
      pdst[pl.ds(i * 16, 16)] = dummy16
    spill_batch(0, tot)
    drain_spills(2)

    cbuf = stages[0][2]
    cbuf[pl.ds(0, 16)] = jnp.full((16,), tot, jnp.int32)
    pltpu.sync_copy(cbuf, counts_hbm.at[pl.ds(wid1 * 16, 16)])

    obase = s * n + base

    @pl.when(r < _NS - 1)
    def _w0():
      pltpu.sync_copy(acc.at[pl.ds(0, nr)], out_hbm.at[pl.ds(obase, nr)])

    @pl.when(r == _NS - 1)
    def _w1():
      pltpu.sync_copy(acc.at[pl.ds(0, nr_last)],
                      out_hbm.at[pl.ds(obase, nr_last)])

  return pl.kernel(
      body,
      out_type=(
          jax.ShapeDtypeStruct((2 * n, f), jnp.float32),
          jax.ShapeDtypeStruct((_NW * cap,), jnp.int32),
          jax.ShapeDtypeStruct((_NW * cap,), jnp.int32),
          jax.ShapeDtypeStruct((_NW * 16,), jnp.int32),
      ),
      mesh=_mesh(),
      scratch_types=[
          pltpu.VMEM((k,), jnp.int32),
          pltpu.VMEM((k,), jnp.int32),
          pltpu.VMEM((k,), jnp.int32),
          pltpu.VMEM((k,), jnp.int32),
          pltpu.VMEM((pcap,), jnp.int32),
          pltpu.VMEM((pcap,), jnp.int32),
          pltpu.VMEM((2048,), jnp.int32),
      ] + [pltpu.VMEM((16,), jnp.int32) for _ in range(4 * _CU)] + [
          pltpu.VMEM((g,), jnp.int32),
          pltpu.VMEM((g, f), jnp.float32),
          pltpu.VMEM((_NR + 1, f), jnp.float32),
          pltpu.SemaphoreType.DMA,
          pltpu.SemaphoreType.DMA,
          pltpu.SemaphoreType.DMA,
          pltpu.SemaphoreType.DMA,
      ],
      compiler_params=pltpu.CompilerParams(needs_layout_passes=False),
  )


def _make_segmax_lists(n, f, cap, g):
  assert f == 256 and g % 16 == 0 and g <= 128
  fh = f // 2
  nr, nr_last = _NR, _NR_LAST

  def body(x_hbm, csrc_hbm, cdst_hbm, counts_hbm, out_hbm,
           ib0, ib1, pd0, pd1, rb0, rb1, cbuf, acc,
           sem_e0, sem_e1, sem_g0, sem_g1):
    wid = lax.axis_index("s") * _NC + lax.axis_index("c")
    parent = wid // 2
    ch = wid - parent * 2
    cbase = ch * fh
    base = parent * nr
    neg = jnp.float32(-jnp.inf)
    sets = ((ib0, pd0, rb0, sem_e0, sem_g0), (ib1, pd1, rb1, sem_e1, sem_g1))

    @pl.loop(0, nr + 1)
    def _init(row):
      for j in range(fh // 16):
        acc[row, pl.ds(j * 16, 16)] = jnp.full((16,), neg, jnp.float32)

    def accum_half(dref, rref, cnt16):
      @pl.loop(0, cnt16 // 16)
      def _grp(u):
        vrow = dref[pl.ds(u * 16, 16)]
        for i in range(16):
          row = vrow[i]
          for j in range(fh // 16):
            acc[row, pl.ds(j * 16, 16)] = jnp.maximum(
                acc[row, pl.ds(j * 16, 16)],
                rref[u * 16 + i, pl.ds(cbase + j * 16, 16)])

    def run_list(lid):
      wb = lid * cap
      pltpu.sync_copy(counts_hbm.at[pl.ds(lid * 16, 16)], cbuf)
      tot = cbuf[pl.ds(0, 16)][0]
      nbt = (tot + g - 1) // g

      def start_edges(i, s):
        ib, pd, _, sem_e, _ = s
        o8 = pl.multiple_of(wb + i * g, 16)
        pltpu.async_copy(csrc_hbm.at[pl.ds(o8, g)], ib, sem_e)
        pltpu.async_copy(cdst_hbm.at[pl.ds(o8, g)], pd, sem_e)

      def wait_edges(s):
        ib, pd, _, sem_e, _ = s
        pltpu.make_async_copy(csrc_hbm.at[pl.ds(0, g)], ib, sem_e).wait()
        pltpu.make_async_copy(cdst_hbm.at[pl.ds(0, g)], pd, sem_e).wait()

      def start_gather(s):
        ib, _, rb, _, sem_g = s
        pltpu.async_copy(x_hbm.at[ib], rb, sem_g)

      def wait_gather(s):
        ib, _, rb, _, sem_g = s
        pltpu.make_async_copy(x_hbm.at[ib], rb, sem_g).wait()

      @pl.when(nbt > 0)
      def _pro():
        start_edges(0, sets[0])
        wait_edges(sets[0])
        start_gather(sets[0])

      @pl.when(nbt > 1)
      def _pro2():
        start_edges(1, sets[1])

      def proc(i, cur, nxt):
        @pl.when(i + 1 < nbt)
        def _nx():
          wait_edges(nxt)
          start_gather(nxt)
        wait_gather(cur)
        accum_half(cur[1], cur[2], g)

        @pl.when(i + 2 < nbt)
        def _pf():
          start_edges(i + 2, cur)

      def pairb(p, _):
        i0 = 2 * p

        @pl.when(i0 < nbt)
        def _a():
          proc(i0, sets[0], sets[1])

        @pl.when(i0 + 1 < nbt)
        def _b():
          proc(i0 + 1, sets[1], sets[0])
        return 0

      lax.fori_loop(0, (nbt + 1) // 2, pairb, 0)

    run_list(parent * 2)
    run_list(parent * 2 + 1)

    @pl.loop(0, nr)
    def _fix(row):
      for j in range(fh // 16):
        sl = pl.ds(j * 16, 16)
        v = acc[row, sl]
        acc[row, sl] = jnp.where(v == neg, jnp.float32(0.0), v)

    @pl.when(parent < _NS - 1)
    def _w0():
      pltpu.sync_copy(acc.at[pl.ds(0, nr), :],
                      out_hbm.at[pl.ds(base, nr), pl.ds(cbase, fh)])

    @pl.when(parent == _NS - 1)
    def _w1():
      pltpu.sync_copy(acc.at[pl.ds(0, nr_last), :],
                      out_hbm.at[pl.ds(base, nr_last), pl.ds(cbase, fh)])

  return pl.kernel(
      body,
      out_type=jax.ShapeDtypeStruct((n, f), jnp.float32),
      mesh=_mesh(),
      scratch_types=[
          pltpu.VMEM((g,), jnp.int32),
          pltpu.VMEM((g,), jnp.int32),
          pltpu.VMEM((g,), jnp.int32),
          pltpu.VMEM((g,), jnp.int32),
          pltpu.VMEM((g, f), jnp.float32),
          pltpu.VMEM((g, f), jnp.float32),
          pltpu.VMEM((16,), jnp.int32),
          pltpu.VMEM((_NR + 1, f // 2), jnp.float32),
          pltpu.SemaphoreType.DMA,
          pltpu.SemaphoreType.DMA,
          pltpu.SemaphoreType.DMA,
          pltpu.SemaphoreType.DMA,
      ],
      compiler_params=pltpu.CompilerParams(needs_layout_passes=False),
  )


def _linear_merge(pa, pb, xr, w_l, b_l, w_r, relu):
  nrows, fin = pa.shape
  fout = w_l.shape[1]
  blk = 1000
  assert nrows % blk == 0

  def body(pa_ref, pb_ref, x_ref, wl_ref, b_ref, wr_ref, o_ref):
    agg = jnp.maximum(pa_ref[...], pb_ref[...])
    agg = jnp.where(jnp.isneginf(agg), 0.0, agg)
    y = jnp.dot(agg, wl_ref[...], preferred_element_type=jnp.float32)
    y = y + jnp.dot(x_ref[...], wr_ref[...], preferred_element_type=jnp.float32)
    y = y + b_ref[...]
    if relu:
      y = jnp.maximum(y, 0.0)
    o_ref[...] = y

  return pl.pallas_call(
      body,
      grid=(nrows // blk,),
      in_specs=[
          pl.BlockSpec((blk, fin), lambda i: (i, 0)),
          pl.BlockSpec((blk, fin), lambda i: (i, 0)),
          pl.BlockSpec((blk, fin), lambda i: (i, 0)),
          pl.BlockSpec((fin, fout), lambda i: (0, 0)),
          pl.BlockSpec((1, fout), lambda i: (0, 0)),
          pl.BlockSpec((fin, fout), lambda i: (0, 0)),
      ],
      out_specs=pl.BlockSpec((blk, fout), lambda i: (i, 0)),
      out_shape=jax.ShapeDtypeStruct((nrows, fout), jnp.float32),
  )(pa, pb, xr, w_l, b_l.reshape(1, fout), w_r)


def _linear(agg, xr, w_l, b_l, w_r, relu):
  nrows, fin = agg.shape
  fout = w_l.shape[1]
  blk = 1000
  assert nrows % blk == 0

  def body(a_ref, x_ref, wl_ref, b_ref, wr_ref, o_ref):
    y = jnp.dot(a_ref[...], wl_ref[...], preferred_element_type=jnp.float32)
    y = y + jnp.dot(x_ref[...], wr_ref[...], preferred_element_type=jnp.float32)
    y = y + b_ref[...]
    if relu:
      y = jnp.maximum(y, 0.0)
    o_ref[...] = y

  return pl.pallas_call(
      body,
      grid=(nrows // blk,),
      in_specs=[
          pl.BlockSpec((blk, fin), lambda i: (i, 0)),
          pl.BlockSpec((blk, fin), lambda i: (i, 0)),
          pl.BlockSpec((fin, fout), lambda i: (0, 0)),
          pl.BlockSpec((1, fout), lambda i: (0, 0)),
          pl.BlockSpec((fin, fout), lambda i: (0, 0)),
      ],
      out_specs=pl.BlockSpec((blk, fout), lambda i: (i, 0)),
      out_shape=jax.ShapeDtypeStruct((nrows, fout), jnp.float32),
  )(agg, xr, w_l, b_l.reshape(1, fout), w_r)


_GA = 128
_CAP = _E // 2 + _GA
_SEG_A = _make_segmax_scan(_N, 128, _E, 1600, _GA)
_SEG_B = _make_segmax_lists(_N, 256, _CAP, 64)


def kernel(x, edge_index, W1_l, b1_l, W1_r, W2_l, b2_l, W2_r):
  src = edge_index[0]
  dst = edge_index[1]
  aggp, csrc, cdst, counts = _SEG_A(x, src, dst, _LUT)
  h = _linear_merge(aggp[:_N], aggp[_N:], x, W1_l, b1_l, W1_r, relu=True)
  agg2 = _SEG_B(h, csrc, cdst, counts)
  return _linear(agg2, h, W2_l, b2_l, W2_r, relu=False)
